# baseline (speedup 1.0000x reference)
import functools
import math

import jax
import jax.numpy as jnp
from jax.experimental import pallas as pl
from jax.experimental.pallas import tpu as pltpu

_FACTOR = 5.0
_EPS = 1e-09


def _prob_sparse_kernel(q_ref, k_ref, v_ref, out_ref, *, k_top, k_pad, L, D):
    q = q_ref[0]
    kk = k_ref[0]
    v = v_ref[0]
    scale = 1.0 / math.sqrt(D)

    qb = q.astype(jnp.bfloat16)
    kb = kk.astype(jnp.bfloat16)

    s = jnp.dot(qb, kb.T, preferred_element_type=jnp.float32) * scale
    m_max = jnp.max(s, axis=-1, keepdims=True)
    m_mean = jnp.sum(s, axis=-1, keepdims=True) * (1.0 / L)
    row = jnp.reshape(m_max - m_mean, (1, L))

    lane_iota = jax.lax.broadcasted_iota(jnp.int32, (1, L), 1)
    neg_inf = jnp.float32(-jnp.inf)

    idx_rows = []
    work = row
    for _ in range(k_top):
        i_j = jnp.argmax(work, axis=-1).reshape(1, 1).astype(jnp.int32)
        idx_rows.append(i_j)
        work = jnp.where(lane_iota == i_j, neg_inf, work)
    for _ in range(k_pad - k_top):
        idx_rows.append(jnp.full((1, 1), -1, dtype=jnp.int32))
    idx_col = jnp.concatenate(idx_rows, axis=0)

    onehot = (jax.lax.broadcasted_iota(jnp.int32, (k_pad, L), 1)
              == idx_col).astype(jnp.bfloat16)

    qs = jnp.dot(onehot, qb, preferred_element_type=jnp.float32)
    ssp = jax.lax.dot_general(
        qs.astype(jnp.bfloat16), kb, (((1,), (1,)), ((), ())),
        preferred_element_type=jnp.float32) * scale
    smax = jnp.max(ssp, axis=-1, keepdims=True)
    e = jnp.exp(ssp - smax)
    att = e / jnp.sum(e, axis=-1, keepdims=True)
    ctx = jnp.dot(att.astype(jnp.bfloat16), v.astype(jnp.bfloat16),
                  preferred_element_type=jnp.float32)

    v_mean = jnp.mean(v, axis=0, keepdims=True)
    delta = ctx - v_mean
    scat = jax.lax.dot_general(
        onehot.astype(jnp.float32), delta, (((0,), (0,)), ((), ())),
        preferred_element_type=jnp.float32)
    out_ref[0] = scat + v_mean


def kernel(Q, K, V):
    B, L, D = Q.shape
    k_top = min(L, max(1, int(_FACTOR * math.log(L + _EPS))))
    k_pad = max(8, ((k_top + 7) // 8) * 8)

    spec = pl.BlockSpec((1, L, D), lambda b: (b, 0, 0))
    return pl.pallas_call(
        functools.partial(_prob_sparse_kernel, k_top=k_top, k_pad=k_pad,
                          L=L, D=D),
        grid=(B,),
        in_specs=[spec, spec, spec],
        out_specs=spec,
        out_shape=jax.ShapeDtypeStruct((B, L, D), jnp.float32),
    )(Q, K, V)


# speedup vs baseline: 1.2077x; 1.2077x over previous
import functools
import math

import jax
import jax.numpy as jnp
from jax.experimental import pallas as pl
from jax.experimental.pallas import tpu as pltpu

_FACTOR = 5.0
_EPS = 1e-09


def _m_kernel(q_ref, k_ref, m_ref, *, L, D):
    q = q_ref[0]
    kk = k_ref[0]
    scale = 1.0 / math.sqrt(D)
    qb = q.astype(jnp.bfloat16)
    kb = kk.astype(jnp.bfloat16)
    s = jnp.dot(qb, kb.T, preferred_element_type=jnp.float32) * scale
    m_max = jnp.max(s, axis=-1, keepdims=True)
    m_mean = jnp.sum(s, axis=-1, keepdims=True) * (1.0 / L)
    m_ref[0] = jnp.reshape(m_max - m_mean, (1, L))


def _attn_kernel(m_ref, q_ref, k_ref, v_ref, out_ref, idx_ref, *,
                 k_top, k_pad, B, L, D):
    b = pl.program_id(0)
    scale = 1.0 / math.sqrt(D)

    @pl.when(b == 0)
    def _topk():
        work = jnp.reshape(m_ref[...], (B, L))
        lane_iota = jax.lax.broadcasted_iota(jnp.int32, (B, L), 1)
        neg_inf = jnp.float32(-jnp.inf)
        cols = []
        for _ in range(k_top):
            i_j = jnp.argmax(work, axis=-1, keepdims=True).astype(jnp.int32)
            cols.append(i_j)
            work = jnp.where(lane_iota == i_j, neg_inf, work)
        for _ in range(k_pad - k_top):
            cols.append(jnp.full((B, 1), -1, dtype=jnp.int32))
        idx_all = jnp.concatenate(cols, axis=1)
        idx_ref[...] = idx_all.T

    q = q_ref[0]
    kk = k_ref[0]
    v = v_ref[0]
    qb = q.astype(jnp.bfloat16)
    kb = kk.astype(jnp.bfloat16)

    idx_all = idx_ref[...]
    b_mask = jax.lax.broadcasted_iota(jnp.int32, (k_pad, B), 1) == b
    idx_col = jnp.sum(jnp.where(b_mask, idx_all, 0), axis=1,
                      keepdims=True)
    onehot = (jax.lax.broadcasted_iota(jnp.int32, (k_pad, L), 1)
              == idx_col).astype(jnp.bfloat16)

    qs = jnp.dot(onehot, qb, preferred_element_type=jnp.float32)
    ssp = jax.lax.dot_general(
        qs.astype(jnp.bfloat16), kb, (((1,), (1,)), ((), ())),
        preferred_element_type=jnp.float32) * scale
    smax = jnp.max(ssp, axis=-1, keepdims=True)
    e = jnp.exp(ssp - smax)
    att = e / jnp.sum(e, axis=-1, keepdims=True)
    ctx = jnp.dot(att.astype(jnp.bfloat16), v.astype(jnp.bfloat16),
                  preferred_element_type=jnp.float32)

    v_mean = jnp.mean(v, axis=0, keepdims=True)
    delta = ctx - v_mean
    scat = jax.lax.dot_general(
        onehot.astype(jnp.float32), delta, (((0,), (0,)), ((), ())),
        preferred_element_type=jnp.float32)
    out_ref[0] = scat + v_mean


def kernel(Q, K, V):
    B, L, D = Q.shape
    k_top = min(L, max(1, int(_FACTOR * math.log(L + _EPS))))
    k_pad = max(8, ((k_top + 7) // 8) * 8)

    spec = pl.BlockSpec((1, L, D), lambda b: (b, 0, 0))
    m_spec = pl.BlockSpec((1, 1, L), lambda b: (b, 0, 0))

    M = pl.pallas_call(
        functools.partial(_m_kernel, L=L, D=D),
        grid=(B,),
        in_specs=[spec, spec],
        out_specs=m_spec,
        out_shape=jax.ShapeDtypeStruct((B, 1, L), jnp.float32),
    )(Q, K)

    return pl.pallas_call(
        functools.partial(_attn_kernel, k_top=k_top, k_pad=k_pad,
                          B=B, L=L, D=D),
        grid=(B,),
        in_specs=[pl.BlockSpec((B, 1, L), lambda b: (0, 0, 0)),
                  spec, spec, spec],
        out_specs=spec,
        out_shape=jax.ShapeDtypeStruct((B, L, D), jnp.float32),
        scratch_shapes=[pltpu.VMEM((k_pad, B), jnp.int32)],
    )(M, Q, K, V)


# speedup vs baseline: 1.6462x; 1.3631x over previous
import functools
import math

import jax
import jax.numpy as jnp
from jax.experimental import pallas as pl
from jax.experimental.pallas import tpu as pltpu

_FACTOR = 5.0
_EPS = 1e-09


def _m_kernel(q_ref, k_ref, m_ref, *, L, D, n_chunks):
    q = q_ref[0]
    kk = k_ref[0]
    scale = 1.0 / math.sqrt(D)
    qt = q.astype(jnp.bfloat16).T
    kb = kk.astype(jnp.bfloat16)
    C = L // n_chunks
    m_acc = jnp.full((1, L), -jnp.inf, dtype=jnp.float32)
    s_acc = jnp.zeros((1, L), dtype=jnp.float32)
    for c in range(n_chunks):
        s_c = jnp.dot(kb[c * C:(c + 1) * C, :], qt,
                      preferred_element_type=jnp.float32)
        m_acc = jnp.maximum(m_acc, jnp.max(s_c, axis=0, keepdims=True))
        s_acc = s_acc + jnp.sum(s_c, axis=0, keepdims=True)
    m_ref[0] = (m_acc - s_acc * (1.0 / L)) * scale


def _attn_kernel(m_ref, q_ref, k_ref, v_ref, out_ref, idx_ref, *,
                 k_top, k_pad, B, L, D):
    b = pl.program_id(0)
    scale = 1.0 / math.sqrt(D)

    @pl.when(b == 0)
    def _topk():
        work = jnp.reshape(m_ref[...], (B, L))
        lane_iota = jax.lax.broadcasted_iota(jnp.int32, (B, L), 1)
        neg_inf = jnp.float32(-jnp.inf)
        cols = []
        for _ in range(k_top):
            i_j = jnp.argmax(work, axis=-1, keepdims=True).astype(jnp.int32)
            cols.append(i_j)
            work = jnp.where(lane_iota == i_j, neg_inf, work)
        for _ in range(k_pad - k_top):
            cols.append(jnp.full((B, 1), -1, dtype=jnp.int32))
        idx_all = jnp.concatenate(cols, axis=1)
        idx_ref[...] = idx_all.T

    q = q_ref[0]
    kk = k_ref[0]
    v = v_ref[0]
    qb = q.astype(jnp.bfloat16)
    kb = kk.astype(jnp.bfloat16)

    idx_all = idx_ref[...]
    b_mask = jax.lax.broadcasted_iota(jnp.int32, (k_pad, B), 1) == b
    idx_col = jnp.sum(jnp.where(b_mask, idx_all, 0), axis=1,
                      keepdims=True)
    onehot = (jax.lax.broadcasted_iota(jnp.int32, (k_pad, L), 1)
              == idx_col).astype(jnp.bfloat16)

    qs = jnp.dot(onehot, qb, preferred_element_type=jnp.float32)
    ssp = jax.lax.dot_general(
        qs.astype(jnp.bfloat16), kb, (((1,), (1,)), ((), ())),
        preferred_element_type=jnp.float32) * scale
    smax = jnp.max(ssp, axis=-1, keepdims=True)
    e = jnp.exp(ssp - smax)
    att = e / jnp.sum(e, axis=-1, keepdims=True)
    ctx = jnp.dot(att.astype(jnp.bfloat16), v.astype(jnp.bfloat16),
                  preferred_element_type=jnp.float32)

    v_mean = jnp.mean(v, axis=0, keepdims=True)
    delta = ctx - v_mean
    scat = jax.lax.dot_general(
        onehot.astype(jnp.float32), delta, (((0,), (0,)), ((), ())),
        preferred_element_type=jnp.float32)
    out_ref[0] = scat + v_mean


def kernel(Q, K, V):
    B, L, D = Q.shape
    k_top = min(L, max(1, int(_FACTOR * math.log(L + _EPS))))
    k_pad = max(8, ((k_top + 7) // 8) * 8)

    spec = pl.BlockSpec((1, L, D), lambda b: (b, 0, 0))
    m_spec = pl.BlockSpec((1, 1, L), lambda b: (b, 0, 0))

    M = pl.pallas_call(
        functools.partial(_m_kernel, L=L, D=D, n_chunks=8),
        grid=(B,),
        in_specs=[spec, spec],
        out_specs=m_spec,
        out_shape=jax.ShapeDtypeStruct((B, 1, L), jnp.float32),
    )(Q, K)

    return pl.pallas_call(
        functools.partial(_attn_kernel, k_top=k_top, k_pad=k_pad,
                          B=B, L=L, D=D),
        grid=(B,),
        in_specs=[pl.BlockSpec((B, 1, L), lambda b: (0, 0, 0)),
                  spec, spec, spec],
        out_specs=spec,
        out_shape=jax.ShapeDtypeStruct((B, L, D), jnp.float32),
        scratch_shapes=[pltpu.VMEM((k_pad, B), jnp.int32)],
    )(M, Q, K, V)


# speedup vs baseline: 1.6482x; 1.0012x over previous
import functools
import math

import jax
import jax.numpy as jnp
from jax.experimental import pallas as pl
from jax.experimental.pallas import tpu as pltpu

_FACTOR = 5.0
_EPS = 1e-09


def _m_kernel(q_ref, k_ref, m_ref, *, L, D, n_chunks):
    q = q_ref[0]
    kk = k_ref[0]
    scale = 1.0 / math.sqrt(D)
    qt = q.astype(jnp.bfloat16).T
    kb = kk.astype(jnp.bfloat16)
    C = L // n_chunks
    m_acc = jnp.full((1, L), -jnp.inf, dtype=jnp.float32)
    s_acc = jnp.zeros((1, L), dtype=jnp.float32)
    for c in range(n_chunks):
        s_c = jnp.dot(kb[c * C:(c + 1) * C, :], qt,
                      preferred_element_type=jnp.float32)
        m_acc = jnp.maximum(m_acc, jnp.max(s_c, axis=0, keepdims=True))
        s_acc = s_acc + jnp.sum(s_c, axis=0, keepdims=True)
    m_ref[0] = (m_acc - s_acc * (1.0 / L)) * scale


def _attn_kernel(m_ref, q_ref, k_ref, v_ref, out_ref, idx_ref, *,
                 k_top, k_pad, B, L, D):
    b = pl.program_id(0)
    scale = 1.0 / math.sqrt(D)

    @pl.when(b == 0)
    def _topk():
        work = jnp.reshape(m_ref[...], (B, L))
        lane_iota = jax.lax.broadcasted_iota(jnp.int32, (B, L), 1)
        neg_inf = jnp.float32(-jnp.inf)
        cols = []
        for _ in range(k_top):
            i_j = jnp.argmax(work, axis=-1, keepdims=True).astype(jnp.int32)
            cols.append(i_j)
            work = jnp.where(lane_iota == i_j, neg_inf, work)
        for _ in range(k_pad - k_top):
            cols.append(jnp.full((B, 1), -1, dtype=jnp.int32))
        idx_all = jnp.concatenate(cols, axis=1)
        idx_ref[...] = idx_all.T

    q = q_ref[0]
    kk = k_ref[0]
    v = v_ref[0]
    qb = q.astype(jnp.bfloat16)
    kb = kk.astype(jnp.bfloat16)

    idx_all = idx_ref[...]
    b_mask = jax.lax.broadcasted_iota(jnp.int32, (k_pad, B), 1) == b
    idx_col = jnp.sum(jnp.where(b_mask, idx_all, 0), axis=1,
                      keepdims=True)
    onehot = (jax.lax.broadcasted_iota(jnp.int32, (k_pad, L), 1)
              == idx_col).astype(jnp.bfloat16)

    qs = jnp.dot(onehot, qb, preferred_element_type=jnp.float32)
    ssp = jax.lax.dot_general(
        qs.astype(jnp.bfloat16), kb, (((1,), (1,)), ((), ())),
        preferred_element_type=jnp.float32) * scale
    smax = jnp.max(ssp, axis=-1, keepdims=True)
    e = jnp.exp(ssp - smax)
    att = e / jnp.sum(e, axis=-1, keepdims=True)
    ctx = jnp.dot(att.astype(jnp.bfloat16), v.astype(jnp.bfloat16),
                  preferred_element_type=jnp.float32)

    v_mean = jnp.mean(v, axis=0, keepdims=True)
    delta = ctx - v_mean
    scat = jax.lax.dot_general(
        onehot.astype(jnp.float32), delta, (((0,), (0,)), ((), ())),
        preferred_element_type=jnp.float32)
    out_ref[0] = scat + v_mean


def kernel(Q, K, V):
    B, L, D = Q.shape
    k_top = min(L, max(1, int(_FACTOR * math.log(L + _EPS))))
    k_pad = max(8, ((k_top + 7) // 8) * 8)

    spec = pl.BlockSpec((1, L, D), lambda b: (b, 0, 0))
    m_spec = pl.BlockSpec((1, 1, L), lambda b: (b, 0, 0))

    M = pl.pallas_call(
        functools.partial(_m_kernel, L=L, D=D, n_chunks=8),
        grid=(B,),
        in_specs=[spec, spec],
        out_specs=m_spec,
        out_shape=jax.ShapeDtypeStruct((B, 1, L), jnp.float32),
        compiler_params=pltpu.CompilerParams(
            dimension_semantics=("parallel",)),
    )(Q, K)

    return pl.pallas_call(
        functools.partial(_attn_kernel, k_top=k_top, k_pad=k_pad,
                          B=B, L=L, D=D),
        grid=(B,),
        in_specs=[pl.BlockSpec((B, 1, L), lambda b: (0, 0, 0)),
                  spec, spec, spec],
        out_specs=spec,
        out_shape=jax.ShapeDtypeStruct((B, L, D), jnp.float32),
        scratch_shapes=[pltpu.VMEM((k_pad, B), jnp.int32)],
    )(M, Q, K, V)


# speedup vs baseline: 1.7888x; 1.0853x over previous
import functools
import math

import jax
import jax.numpy as jnp
from jax.experimental import pallas as pl
from jax.experimental.pallas import tpu as pltpu

_FACTOR = 5.0
_EPS = 1e-09


def _fused_kernel(q_hbm, k_hbm, v_hbm, out_ref,
                  qs_ref, ks_ref, vs_ref, m_ref, vsum_ref, idx_ref, *,
                  k_top, k_pad, B, L, D, n_chunks):
    g = pl.program_id(0)
    scale = 1.0 / math.sqrt(D)

    @pl.when(g < B)
    def _phase1():
        q = q_hbm[0]
        kk = k_hbm[0]
        v = v_hbm[0]
        qb = q.astype(jnp.bfloat16)
        kb = kk.astype(jnp.bfloat16)
        qs_ref[g] = qb
        ks_ref[g] = kb
        vs_ref[g] = v.astype(jnp.bfloat16)
        vsum_ref[g] = jnp.sum(v, axis=0, keepdims=True)

        qt = qb.T
        C = L // n_chunks
        m_acc = jnp.full((1, L), -jnp.inf, dtype=jnp.float32)
        s_acc = jnp.zeros((1, L), dtype=jnp.float32)
        for c in range(n_chunks):
            s_c = jnp.dot(kb[c * C:(c + 1) * C, :], qt,
                          preferred_element_type=jnp.float32)
            m_acc = jnp.maximum(m_acc, jnp.max(s_c, axis=0, keepdims=True))
            s_acc = s_acc + jnp.sum(s_c, axis=0, keepdims=True)
        m_ref[g] = (m_acc - s_acc * (1.0 / L)) * scale

    @pl.when(g == B)
    def _topk():
        work = jnp.reshape(m_ref[...], (B, L))
        lane_iota = jax.lax.broadcasted_iota(jnp.int32, (B, L), 1)
        neg_inf = jnp.float32(-jnp.inf)
        cols = []
        for _ in range(k_top):
            i_j = jnp.argmax(work, axis=-1, keepdims=True).astype(jnp.int32)
            cols.append(i_j)
            work = jnp.where(lane_iota == i_j, neg_inf, work)
        for _ in range(k_pad - k_top):
            cols.append(jnp.full((B, 1), -1, dtype=jnp.int32))
        idx_ref[...] = jnp.concatenate(cols, axis=1).T

    @pl.when(g >= B)
    def _phase2():
        b = g - B
        qb = qs_ref[b]
        kb = ks_ref[b]
        vb = vs_ref[b]

        idx_all = idx_ref[...]
        b_mask = jax.lax.broadcasted_iota(jnp.int32, (k_pad, B), 1) == b
        idx_col = jnp.sum(jnp.where(b_mask, idx_all, 0), axis=1,
                          keepdims=True)
        onehot = (jax.lax.broadcasted_iota(jnp.int32, (k_pad, L), 1)
                  == idx_col).astype(jnp.bfloat16)

        qsel = jnp.dot(onehot, qb, preferred_element_type=jnp.float32)
        ssp = jax.lax.dot_general(
            qsel.astype(jnp.bfloat16), kb, (((1,), (1,)), ((), ())),
            preferred_element_type=jnp.float32) * scale
        smax = jnp.max(ssp, axis=-1, keepdims=True)
        e = jnp.exp(ssp - smax)
        att = e / jnp.sum(e, axis=-1, keepdims=True)
        ctx = jnp.dot(att.astype(jnp.bfloat16), vb,
                      preferred_element_type=jnp.float32)

        v_mean = vsum_ref[b] * (1.0 / L)
        delta = ctx - v_mean
        scat = jax.lax.dot_general(
            onehot.astype(jnp.float32), delta, (((0,), (0,)), ((), ())),
            preferred_element_type=jnp.float32)
        out_ref[0] = scat + v_mean


def kernel(Q, K, V):
    B, L, D = Q.shape
    k_top = min(L, max(1, int(_FACTOR * math.log(L + _EPS))))
    k_pad = max(8, ((k_top + 7) // 8) * 8)

    in_spec = pl.BlockSpec((1, L, D), lambda g: (jnp.minimum(g, B - 1), 0, 0))
    out_spec = pl.BlockSpec((1, L, D), lambda g: (jnp.maximum(g - B, 0), 0, 0))

    return pl.pallas_call(
        functools.partial(_fused_kernel, k_top=k_top, k_pad=k_pad,
                          B=B, L=L, D=D, n_chunks=8),
        grid=(2 * B,),
        in_specs=[in_spec, in_spec, in_spec],
        out_specs=out_spec,
        out_shape=jax.ShapeDtypeStruct((B, L, D), jnp.float32),
        scratch_shapes=[
            pltpu.VMEM((B, L, D), jnp.bfloat16),
            pltpu.VMEM((B, L, D), jnp.bfloat16),
            pltpu.VMEM((B, L, D), jnp.bfloat16),
            pltpu.VMEM((B, 1, L), jnp.float32),
            pltpu.VMEM((B, 1, D), jnp.float32),
            pltpu.VMEM((k_pad, B), jnp.int32),
        ],
    )(Q, K, V)


# speedup vs baseline: 1.7956x; 1.0038x over previous
import functools
import math

import jax
import jax.numpy as jnp
from jax.experimental import pallas as pl
from jax.experimental.pallas import tpu as pltpu

_FACTOR = 5.0
_EPS = 1e-09


def _fused_kernel(q_hbm, k_hbm, v_hbm, out_ref,
                  qs_ref, ks_ref, vs_ref, m_ref, vsum_ref, idx_ref, *,
                  k_top, k_pad, B, L, D, n_chunks):
    g = pl.program_id(0)
    scale = 1.0 / math.sqrt(D)

    @pl.when(g < B)
    def _phase1():
        q = q_hbm[0]
        kk = k_hbm[0]
        v = v_hbm[0]
        qb = q.astype(jnp.bfloat16)
        kb = kk.astype(jnp.bfloat16)
        qs_ref[g] = qb
        ks_ref[g] = kb
        vs_ref[g] = v.astype(jnp.bfloat16)
        vsum_ref[g] = jnp.sum(v, axis=0, keepdims=True)

        qt = qb.T
        C = L // n_chunks
        m_acc = jnp.full((1, L), -jnp.inf, dtype=jnp.float32)
        s_acc = jnp.zeros((1, L), dtype=jnp.float32)
        for c in range(n_chunks):
            s_c = jnp.dot(kb[c * C:(c + 1) * C, :], qt,
                          preferred_element_type=jnp.float32)
            m_acc = jnp.maximum(m_acc, jnp.max(s_c, axis=0, keepdims=True))
            s_acc = s_acc + jnp.sum(s_c, axis=0, keepdims=True)
        m_ref[g] = (m_acc - s_acc * (1.0 / L)) * scale

    @pl.when(g == B)
    def _topk():
        work = jnp.reshape(m_ref[...], (B, L))
        lane_iota = jax.lax.broadcasted_iota(jnp.int32, (B, L), 1)
        neg_inf = jnp.float32(-jnp.inf)
        cols = []
        for _ in range(k_top):
            i_j = jnp.argmax(work, axis=-1, keepdims=True).astype(jnp.int32)
            cols.append(i_j)
            work = jnp.where(lane_iota == i_j, neg_inf, work)
        for _ in range(k_pad - k_top):
            cols.append(jnp.full((B, 1), -1, dtype=jnp.int32))
        idx_ref[...] = jnp.concatenate(cols, axis=1).T

    @pl.when(g >= B)
    def _phase2():
        b = g - B
        qb = qs_ref[b]
        kb = ks_ref[b]
        vb = vs_ref[b]

        idx_all = idx_ref[...]
        b_mask = jax.lax.broadcasted_iota(jnp.int32, (k_pad, B), 1) == b
        idx_col = jnp.sum(jnp.where(b_mask, idx_all, 0), axis=1,
                          keepdims=True)
        onehot = (jax.lax.broadcasted_iota(jnp.int32, (k_pad, L), 1)
                  == idx_col).astype(jnp.bfloat16)

        qsel = jnp.dot(onehot, qb, preferred_element_type=jnp.float32)
        ssp = jax.lax.dot_general(
            qsel.astype(jnp.bfloat16), kb, (((1,), (1,)), ((), ())),
            preferred_element_type=jnp.float32) * scale
        smax = jnp.max(ssp, axis=-1, keepdims=True)
        e = jnp.exp(ssp - smax)
        att = e / jnp.sum(e, axis=-1, keepdims=True)
        ctx = jnp.dot(att.astype(jnp.bfloat16), vb,
                      preferred_element_type=jnp.float32)

        v_mean = vsum_ref[b] * (1.0 / L)
        delta = ctx - v_mean
        scat = jax.lax.dot_general(
            onehot.astype(jnp.float32), delta, (((0,), (0,)), ((), ())),
            preferred_element_type=jnp.float32)
        out_ref[0] = scat + v_mean


def kernel(Q, K, V):
    B, L, D = Q.shape
    k_top = min(L, max(1, int(_FACTOR * math.log(L + _EPS))))
    k_pad = max(8, ((k_top + 7) // 8) * 8)

    in_spec = pl.BlockSpec((1, L, D), lambda g: (jnp.minimum(g, B - 1), 0, 0))
    out_spec = pl.BlockSpec((1, L, D), lambda g: (jnp.maximum(g - B, 0), 0, 0))

    return pl.pallas_call(
        functools.partial(_fused_kernel, k_top=k_top, k_pad=k_pad,
                          B=B, L=L, D=D, n_chunks=4),
        grid=(2 * B,),
        in_specs=[in_spec, in_spec, in_spec],
        out_specs=out_spec,
        out_shape=jax.ShapeDtypeStruct((B, L, D), jnp.float32),
        scratch_shapes=[
            pltpu.VMEM((B, L, D), jnp.bfloat16),
            pltpu.VMEM((B, L, D), jnp.bfloat16),
            pltpu.VMEM((B, L, D), jnp.bfloat16),
            pltpu.VMEM((B, 1, L), jnp.float32),
            pltpu.VMEM((B, 1, D), jnp.float32),
            pltpu.VMEM((k_pad, B), jnp.int32),
        ],
    )(Q, K, V)
